# baseline (speedup 1.0000x reference)
import functools

import jax
import jax.numpy as jnp
import numpy as np
from jax import lax
from jax.experimental import pallas as pl
from jax.experimental.pallas import tpu as pltpu
from jax.experimental.pallas import tpu_sc as plsc

N_NODES = 100000
MAX_DEGREE = 64
BATCH = 1024
NS1 = 10
NS2 = 25

_info = plsc.get_sparse_core_info()
NC, NSUB, LANES = _info.num_cores, _info.num_subcores, _info.num_lanes
NW = NC * NSUB
IDS_W = BATCH // NW
F1_W = IDS_W * NS1
F2_W = F1_W * NS2
GCHUNKS = 4
GCH = F1_W // GCHUNKS

PER1 = np.lcm(16, NS1)
PER2 = np.lcm(16, NS2)
ROWS_PER1 = PER1 // NS1
ROWS_PER2 = PER2 // NS2

_P1 = np.arange(PER1)
_P2 = np.arange(PER2)
_PM1 = (_P1 // NS1).astype(np.int32)
_PM2 = (_P2 // NS2).astype(np.int32)


def _sample_body(inputs_hbm, adj_hbm, pm1_hbm, pc1_hbm, pm2_hbm, pc2_hbm,
                 out1_hbm, out2_hbm,
                 pm1_v, pc1_v, pm2_v, pc2_v, ids_v, rows1_v, f1_v, rows2_v,
                 f2_v, sem):
    wid = lax.axis_index("s") * NC + lax.axis_index("c")

    pltpu.sync_copy(pm1_hbm, pm1_v)
    pltpu.sync_copy(pc1_hbm, pc1_v)
    pltpu.sync_copy(pm2_hbm, pm2_v)
    pltpu.sync_copy(pc2_hbm, pc2_v)
    base = pl.multiple_of(wid * IDS_W, IDS_W)
    pltpu.sync_copy(inputs_hbm.at[pl.ds(base, IDS_W)], ids_v)

    pltpu.async_copy(adj_hbm.at[ids_v], rows1_v, sem).wait()

    for g in range(F1_W // PER1):
        for c in range(PER1 // LANES):
            m = g * ROWS_PER1 + pm1_v[pl.ds(c * LANES, LANES)]
            cv = pc1_v[pl.ds(c * LANES, LANES)]
            val = plsc.load_gather(rows1_v, [m, cv])
            f1_v[g, pl.ds(c * LANES, LANES)] = val

    for j in range(GCHUNKS):
        pltpu.sync_copy(f1_v.at[j],
                        out1_hbm.at[pl.ds(pl.multiple_of(wid * F1_W + j * GCH, GCH),
                                          GCH)])

    copies = [
        pltpu.async_copy(adj_hbm.at[f1_v.at[j]],
                         rows2_v.at[pl.ds(j * GCH, GCH)], sem)
        for j in range(GCHUNKS)
    ]
    for cp in copies:
        cp.wait()

    def sel2(g, carry):
        for c in range(PER2 // LANES):
            m = g * ROWS_PER2 + pm2_v[pl.ds(c * LANES, LANES)]
            cv = pc2_v[pl.ds(c * LANES, LANES)]
            val = plsc.load_gather(rows2_v, [m, cv])
            f2_v[pl.ds(g * PER2 + c * LANES, LANES)] = val
        return carry

    lax.fori_loop(0, F2_W // PER2, sel2, 0)

    pltpu.sync_copy(f2_v, out2_hbm.at[pl.ds(pl.multiple_of(wid * F2_W, F2_W),
                                            F2_W)])


@functools.partial(
    pl.kernel,
    mesh=plsc.VectorSubcoreMesh(core_axis_name="c", subcore_axis_name="s"),
    out_type=(jax.ShapeDtypeStruct((BATCH * NS1,), jnp.int32),
              jax.ShapeDtypeStruct((BATCH * NS1 * NS2,), jnp.int32)),
    scratch_types=[
        pltpu.VMEM((PER1,), jnp.int32),
        pltpu.VMEM((PER1,), jnp.int32),
        pltpu.VMEM((PER2,), jnp.int32),
        pltpu.VMEM((PER2,), jnp.int32),
        pltpu.VMEM((IDS_W,), jnp.int32),
        pltpu.VMEM((IDS_W, MAX_DEGREE), jnp.int32),
        pltpu.VMEM((GCHUNKS, GCH), jnp.int32),
        pltpu.VMEM((F1_W, MAX_DEGREE), jnp.int32),
        pltpu.VMEM((F2_W,), jnp.int32),
        pltpu.SemaphoreType.DMA,
    ],
    compiler_params=pltpu.CompilerParams(use_tc_tiling_on_sc=False,
                                         needs_layout_passes=False),
)
def _sample_kernel(inputs_hbm, adj_hbm, pm1_hbm, pc1_hbm, pm2_hbm, pc2_hbm,
                   out1_hbm, out2_hbm, *scratch):
    _sample_body(inputs_hbm, adj_hbm, pm1_hbm, pc1_hbm, pm2_hbm, pc2_hbm,
                 out1_hbm, out2_hbm, *scratch)


def kernel(inputs, adj_info):
    key = jax.random.key(42)
    key, sub1 = jax.random.split(key)
    perm1 = jax.random.permutation(sub1, MAX_DEGREE).astype(jnp.int32)
    key, sub2 = jax.random.split(key)
    perm2 = jax.random.permutation(sub2, MAX_DEGREE).astype(jnp.int32)
    pc1 = jnp.take(perm1[:NS1], jnp.asarray(_P1 % NS1, dtype=jnp.int32))
    pc2 = jnp.take(perm2[:NS2], jnp.asarray(_P2 % NS2, dtype=jnp.int32))
    pm1 = jnp.asarray(_PM1)
    pm2 = jnp.asarray(_PM2)

    out1, out2 = _sample_kernel(inputs, adj_info, pm1, pc1, pm2, pc2)
    return (inputs, out1, out2)


# speedup vs baseline: 1.0127x; 1.0127x over previous
import functools

import jax
import jax.numpy as jnp
import numpy as np
from jax import lax
from jax.experimental import pallas as pl
from jax.experimental.pallas import tpu as pltpu
from jax.experimental.pallas import tpu_sc as plsc

N_NODES = 100000
MAX_DEGREE = 64
BATCH = 1024
NS1 = 10
NS2 = 25

_info = plsc.get_sparse_core_info()
NC, NSUB, LANES = _info.num_cores, _info.num_subcores, _info.num_lanes
NW = NC * NSUB
IDS_W = BATCH // NW
F1_W = IDS_W * NS1
F2_W = F1_W * NS2
GCHUNKS = 4
GCH = F1_W // GCHUNKS

PER1 = np.lcm(16, NS1)
PER2 = np.lcm(16, NS2)
ROWS_PER1 = PER1 // NS1
ROWS_PER2 = PER2 // NS2

_P1 = np.arange(PER1)
_P2 = np.arange(PER2)
_PM1 = (_P1 // NS1).astype(np.int32)
_PM2 = (_P2 // NS2).astype(np.int32)


def _sample_body(inputs_hbm, adj_hbm, pm1_hbm, pc1_hbm, pm2_hbm, pc2_hbm,
                 out0_hbm, out1_hbm, out2_hbm,
                 pm1_v, pc1_v, pm2_v, pc2_v, ids_v, rows1_v, f1_v, rows2_v,
                 f2_v, sem):
    wid = lax.axis_index("s") * NC + lax.axis_index("c")

    pltpu.sync_copy(pm1_hbm, pm1_v)
    pltpu.sync_copy(pc1_hbm, pc1_v)
    pltpu.sync_copy(pm2_hbm, pm2_v)
    pltpu.sync_copy(pc2_hbm, pc2_v)
    base = pl.multiple_of(wid * IDS_W, IDS_W)
    pltpu.sync_copy(inputs_hbm.at[pl.ds(base, IDS_W)], ids_v)

    cp1 = pltpu.async_copy(adj_hbm.at[ids_v], rows1_v, sem)
    pltpu.sync_copy(ids_v, out0_hbm.at[pl.ds(base, IDS_W)])
    cp1.wait()

    for g in range(F1_W // PER1):
        for c in range(PER1 // LANES):
            m = g * ROWS_PER1 + pm1_v[pl.ds(c * LANES, LANES)]
            cv = pc1_v[pl.ds(c * LANES, LANES)]
            val = plsc.load_gather(rows1_v, [m, cv])
            f1_v[g, pl.ds(c * LANES, LANES)] = val

    for j in range(GCHUNKS):
        pltpu.sync_copy(f1_v.at[j],
                        out1_hbm.at[pl.ds(pl.multiple_of(wid * F1_W + j * GCH, GCH),
                                          GCH)])

    copies = [
        pltpu.async_copy(adj_hbm.at[f1_v.at[j]],
                         rows2_v.at[pl.ds(j * GCH, GCH)], sem)
        for j in range(GCHUNKS)
    ]
    for cp in copies:
        cp.wait()

    def sel2(g, carry):
        for c in range(PER2 // LANES):
            m = g * ROWS_PER2 + pm2_v[pl.ds(c * LANES, LANES)]
            cv = pc2_v[pl.ds(c * LANES, LANES)]
            val = plsc.load_gather(rows2_v, [m, cv])
            f2_v[pl.ds(g * PER2 + c * LANES, LANES)] = val
        return carry

    lax.fori_loop(0, F2_W // PER2, sel2, 0)

    pltpu.sync_copy(f2_v, out2_hbm.at[pl.ds(pl.multiple_of(wid * F2_W, F2_W),
                                            F2_W)])


@functools.partial(
    pl.kernel,
    mesh=plsc.VectorSubcoreMesh(core_axis_name="c", subcore_axis_name="s"),
    out_type=(jax.ShapeDtypeStruct((BATCH,), jnp.int32),
              jax.ShapeDtypeStruct((BATCH * NS1,), jnp.int32),
              jax.ShapeDtypeStruct((BATCH * NS1 * NS2,), jnp.int32)),
    scratch_types=[
        pltpu.VMEM((PER1,), jnp.int32),
        pltpu.VMEM((PER1,), jnp.int32),
        pltpu.VMEM((PER2,), jnp.int32),
        pltpu.VMEM((PER2,), jnp.int32),
        pltpu.VMEM((IDS_W,), jnp.int32),
        pltpu.VMEM((IDS_W, MAX_DEGREE), jnp.int32),
        pltpu.VMEM((GCHUNKS, GCH), jnp.int32),
        pltpu.VMEM((F1_W, MAX_DEGREE), jnp.int32),
        pltpu.VMEM((F2_W,), jnp.int32),
        pltpu.SemaphoreType.DMA,
    ],
    compiler_params=pltpu.CompilerParams(use_tc_tiling_on_sc=False,
                                         needs_layout_passes=False),
)
def _sample_kernel(inputs_hbm, adj_hbm, pm1_hbm, pc1_hbm, pm2_hbm, pc2_hbm,
                   out0_hbm, out1_hbm, out2_hbm, *scratch):
    _sample_body(inputs_hbm, adj_hbm, pm1_hbm, pc1_hbm, pm2_hbm, pc2_hbm,
                 out0_hbm, out1_hbm, out2_hbm, *scratch)


def kernel(inputs, adj_info):
    key = jax.random.key(42)
    key, sub1 = jax.random.split(key)
    perm1 = jax.random.permutation(sub1, MAX_DEGREE).astype(jnp.int32)
    key, sub2 = jax.random.split(key)
    perm2 = jax.random.permutation(sub2, MAX_DEGREE).astype(jnp.int32)
    pc1 = jnp.take(perm1[:NS1], jnp.asarray(_P1 % NS1, dtype=jnp.int32))
    pc2 = jnp.take(perm2[:NS2], jnp.asarray(_P2 % NS2, dtype=jnp.int32))
    pm1 = jnp.asarray(_PM1)
    pm2 = jnp.asarray(_PM2)

    out0, out1, out2 = _sample_kernel(inputs, adj_info, pm1, pc1, pm2, pc2)
    return (out0, out1, out2)


# speedup vs baseline: 1.8840x; 1.8603x over previous
import functools

import jax
import jax.numpy as jnp
import numpy as np
from jax import lax
from jax.experimental import pallas as pl
from jax.experimental.pallas import tpu as pltpu
from jax.experimental.pallas import tpu_sc as plsc

N_NODES = 100000
MAX_DEGREE = 64
BATCH = 1024
NS1 = 10
NS2 = 25

_info = plsc.get_sparse_core_info()
NC, NSUB, LANES = _info.num_cores, _info.num_subcores, _info.num_lanes
NW = NC * NSUB
IDS_W = BATCH // NW
F1_W = IDS_W * NS1
F2_W = F1_W * NS2
GCH = 80
G1 = F1_W // GCH
G2 = F2_W // GCH

PER1 = int(np.lcm(16, NS1))
PER2 = int(np.lcm(16, NS2))
ROWS_PER2 = PER2 // NS2

_key = jax.random.key(42)
_key, _sub1 = jax.random.split(_key)
_PERM1 = np.asarray(jax.random.permutation(_sub1, MAX_DEGREE))
_key, _sub2 = jax.random.split(_key)
_PERM2 = np.asarray(jax.random.permutation(_sub2, MAX_DEGREE))

_P1 = np.arange(PER1)
_P2 = np.arange(PER2)
_PM1 = (_P1 // NS1).astype(np.int32)
_PM2 = (_P2 // NS2).astype(np.int32)
_PC1F = (_PERM1[:NS1][_P1 % NS1] * N_NODES).astype(np.int32)
_PC2F = (_PERM2[:NS2][_P2 % NS2] * N_NODES).astype(np.int32)


def _body(inputs_hbm, adjf_hbm, pm1_hbm, pc1_hbm, pm2_hbm, pc2_hbm,
          out0_hbm, out1_hbm, out2_hbm,
          pm1_v, pc1_v, pm2_v, pc2_v, ids_v, idx1_v, f1_v, idx2_v, f2_v,
          sem, sem2):
    wid = lax.axis_index("s") * NC + lax.axis_index("c")

    pltpu.sync_copy(pm1_hbm, pm1_v)
    pltpu.sync_copy(pc1_hbm, pc1_v)
    pltpu.sync_copy(pm2_hbm, pm2_v)
    pltpu.sync_copy(pc2_hbm, pc2_v)
    base = pl.multiple_of(wid * IDS_W, IDS_W)
    pltpu.sync_copy(inputs_hbm.at[pl.ds(base, IDS_W)], ids_v)
    cp0 = pltpu.async_copy(ids_v, out0_hbm.at[pl.ds(base, IDS_W)], sem2)

    for i in range(F1_W // LANES):
        m = pm1_v[pl.ds((i % (PER1 // LANES)) * LANES, LANES)] \
            + (i // (PER1 // LANES)) * (PER1 // NS1)
        seed = plsc.load_gather(ids_v, [m])
        cf = pc1_v[pl.ds((i % (PER1 // LANES)) * LANES, LANES)]
        idx1_v[i // (GCH // LANES),
               pl.ds((i % (GCH // LANES)) * LANES, LANES)] = seed + cf

    cps1 = [
        pltpu.async_copy(adjf_hbm.at[idx1_v.at[j]],
                         f1_v.at[pl.ds(j * GCH, GCH)], sem)
        for j in range(G1)
    ]
    for cp in cps1:
        cp.wait()

    cpo1 = pltpu.async_copy(
        f1_v, out1_hbm.at[pl.ds(pl.multiple_of(wid * F1_W, F1_W), F1_W)], sem2)

    def grp(g, carry):
        for c in range(PER2 // LANES):
            m = pm2_v[pl.ds(c * LANES, LANES)] + g * ROWS_PER2
            node = plsc.load_gather(f1_v, [m])
            cf = pc2_v[pl.ds(c * LANES, LANES)]
            idx2_v[g * (PER2 // GCH) + c // (GCH // LANES),
                   pl.ds((c % (GCH // LANES)) * LANES, LANES)] = node + cf
        for j in range(PER2 // GCH):
            row = g * (PER2 // GCH) + j
            pltpu.async_copy(adjf_hbm.at[idx2_v.at[row]],
                             f2_v.at[pl.ds(g * PER2 + j * GCH, GCH)], sem)
        return carry

    lax.fori_loop(0, F2_W // PER2, grp, 0)

    pltpu.make_async_copy(out2_hbm.at[pl.ds(0, F2_W)], f2_v, sem).wait()

    pltpu.sync_copy(f2_v,
                    out2_hbm.at[pl.ds(pl.multiple_of(wid * F2_W, F2_W), F2_W)])
    cp0.wait()
    cpo1.wait()


@functools.partial(
    pl.kernel,
    mesh=plsc.VectorSubcoreMesh(core_axis_name="c", subcore_axis_name="s"),
    out_type=(jax.ShapeDtypeStruct((BATCH,), jnp.int32),
              jax.ShapeDtypeStruct((BATCH * NS1,), jnp.int32),
              jax.ShapeDtypeStruct((BATCH * NS1 * NS2,), jnp.int32)),
    scratch_types=[
        pltpu.VMEM((PER1,), jnp.int32),
        pltpu.VMEM((PER1,), jnp.int32),
        pltpu.VMEM((PER2,), jnp.int32),
        pltpu.VMEM((PER2,), jnp.int32),
        pltpu.VMEM((IDS_W,), jnp.int32),
        pltpu.VMEM((G1, GCH), jnp.int32),
        pltpu.VMEM((F1_W,), jnp.int32),
        pltpu.VMEM((G2, GCH), jnp.int32),
        pltpu.VMEM((F2_W,), jnp.int32),
        pltpu.SemaphoreType.DMA,
        pltpu.SemaphoreType.DMA,
    ],
    compiler_params=pltpu.CompilerParams(use_tc_tiling_on_sc=False,
                                         needs_layout_passes=False),
)
def _sample_kernel(inputs_hbm, adjf_hbm, pm1_hbm, pc1_hbm, pm2_hbm, pc2_hbm,
                   out0_hbm, out1_hbm, out2_hbm, *scratch):
    _body(inputs_hbm, adjf_hbm, pm1_hbm, pc1_hbm, pm2_hbm, pc2_hbm,
          out0_hbm, out1_hbm, out2_hbm, *scratch)


def kernel(inputs, adj_info):
    adjf = jnp.reshape(adj_info.T, (N_NODES * MAX_DEGREE,))
    out0, out1, out2 = _sample_kernel(inputs, adjf,
                                      jnp.asarray(_PM1), jnp.asarray(_PC1F),
                                      jnp.asarray(_PM2), jnp.asarray(_PC2F))
    return (out0, out1, out2)
